# baseline (speedup 1.0000x reference)
import functools

import jax
import jax.numpy as jnp
from jax import lax
from jax.experimental import pallas as pl
from jax.experimental.pallas import tpu as pltpu
from jax.experimental.pallas import tpu_sc as plsc

L = 16
NC, NS = 2, 16
NW = NC * NS
N = 1024 * 200
C = 80
PER_W = N // NW
NCHUNK = PER_W // C

X_D = 32
OUT_D = 153
SPECIES_D = 32
EMB_D = 16

NUM_SPECIES = 2048
NUM_ABILITIES = 512
NUM_ITEMS = 2048
NUM_MOVES = 1024

VOCABS = (NUM_SPECIES, NUM_ABILITIES, NUM_ITEMS,
          NUM_MOVES, NUM_MOVES, NUM_MOVES, NUM_MOVES)
RAW_OFF = 121


def _body(x_hbm, g_hbm, sp_hbm, ab_hbm, it_hbm, mv_hbm, out_hbm,
          ab_t, it_t, mv_t, x_a, x_b, sp_v, out_a, out_b,
          ids0, ids1, ids2, ids3, ids4, ids5, ids6,
          g_v, sp_sh, out_sh, sem_xa, sem_xb, sem_o, sem):
    sid = lax.axis_index("s")
    wid = sid * NC + lax.axis_index("c")
    base0 = wid * PER_W

    pltpu.sync_copy(ab_hbm, ab_t)
    pltpu.sync_copy(it_hbm, it_t)
    pltpu.sync_copy(mv_hbm, mv_t)
    pltpu.sync_copy(g_hbm, g_v)
    zrow = jnp.zeros((L,), jnp.float32)
    ab_t[0, :] = zrow
    it_t[0, :] = zrow
    mv_t[0, :] = zrow
    g = g_v[...]

    srows = NUM_SPECIES // NS
    piece = 64
    for k in range(srows // piece):
        off = sid * srows + k * piece
        pltpu.sync_copy(sp_hbm.at[pl.ds(off, piece)], sp_v.at[pl.ds(0, piece)])
        pltpu.sync_copy(sp_v.at[pl.ds(0, piece)], sp_sh.at[pl.ds(off, piece)])
    plsc.subcore_barrier()

    id_bufs = (ids0, ids1, ids2, ids3, ids4, ids5, ids6)
    iota = lax.broadcasted_iota(jnp.int32, (L,), 0)
    col_tabs = ((1, 32, ab_t), (2, 48, it_t), (3, 64, mv_t),
                (4, 80, mv_t), (5, 96, mv_t), (6, 112, mv_t))

    def x_slice(base):
        return x_hbm.at[pl.ds(base * X_D, C * X_D)]

    def out_slice(base):
        return out_hbm.at[pl.ds(base * OUT_D, C * OUT_D)]

    def phase(j, x_p, x_q, out_p, sem_xp, sem_xq):
        base = base0 + j * C
        pltpu.make_async_copy(x_slice(base), x_p, sem_xp).wait()

        @pl.when(j + 1 < NCHUNK)
        def _():
            pltpu.async_copy(x_slice(base + C), x_q, sem_xq)

        def ids_blk(b, carry2):
            rvec = iota * X_D + b * (L * X_D)
            for col in range(7):
                xc = plsc.load_gather(x_p, [rvec + col])
                idv = jnp.clip((xc + g).astype(jnp.int32), 0, VOCABS[col] - 1)
                id_bufs[col][pl.ds(b * L, L)] = idv
            return carry2

        lax.fori_loop(0, C // L, ids_blk, 0, unroll=True)

        pltpu.async_copy(sp_sh.at[ids0], sp_v, sem)

        pltpu.make_async_copy(sp_sh.at[ids0], sp_v, sem).wait()

        def asm_row(r, carry2):
            ro = r * OUT_D
            rx = r * X_D
            out_p[pl.ds(ro, L)] = sp_v[r, 0:L]
            out_p[pl.ds(ro + L, L)] = sp_v[r, L:SPECIES_D]
            out_p[pl.ds(ro + 128, L)] = x_p[pl.ds(rx + 7, L)] + g
            out_p[pl.ds(ro + 137, L)] = x_p[pl.ds(rx + 16, L)] + g
            return carry2

        lax.fori_loop(0, C, asm_row, 0, unroll=4)

        def asm_blk(b, carry2):
            r_out = (iota + b * L) * OUT_D
            for col, off, tab in col_tabs:
                ids = id_bufs[col][pl.ds(b * L, L)]
                for d in range(EMB_D):
                    v = plsc.load_gather(tab, [ids, jnp.full((L,), d, jnp.int32)])
                    plsc.store_scatter(out_p, [r_out + (off + d)], v)
            return carry2

        lax.fori_loop(0, C // L, asm_blk, 0, unroll=False)

        @pl.when(j >= 1)
        def _():
            pltpu.make_async_copy(
                out_sh.at[sid], out_slice(base - C), sem_o).wait()

        pltpu.sync_copy(out_p, out_sh.at[sid])
        pltpu.async_copy(out_sh.at[sid], out_slice(base), sem_o)

    pltpu.async_copy(x_slice(base0), x_a, sem_xa)

    def pair(k, carry):
        j = 2 * k
        phase(j, x_a, x_b, out_a, sem_xa, sem_xb)
        phase(j + 1, x_b, x_a, out_b, sem_xb, sem_xa)
        return carry

    lax.fori_loop(0, NCHUNK // 2, pair, 0, unroll=False)

    last = base0 + (NCHUNK - 1) * C
    pltpu.make_async_copy(out_sh.at[sid], out_slice(last), sem_o).wait()


@functools.partial(jax.jit)
def _run(xf, g, sp, ab, it, mv):
    mesh = plsc.VectorSubcoreMesh(core_axis_name="c", subcore_axis_name="s")
    f = functools.partial(
        pl.kernel,
        mesh=mesh,
        compiler_params=pltpu.CompilerParams(
            needs_layout_passes=False, use_tc_tiling_on_sc=False),
        out_type=jax.ShapeDtypeStruct((N * OUT_D,), jnp.float32),
        scratch_types=[
            pltpu.VMEM((NUM_ABILITIES, EMB_D), jnp.float32),
            pltpu.VMEM((NUM_ITEMS, EMB_D), jnp.float32),
            pltpu.VMEM((NUM_MOVES, EMB_D), jnp.float32),
            pltpu.VMEM((C * X_D,), jnp.float32),
            pltpu.VMEM((C * X_D,), jnp.float32),
            pltpu.VMEM((C, SPECIES_D), jnp.float32),
            pltpu.VMEM((C * OUT_D,), jnp.float32),
            pltpu.VMEM((C * OUT_D,), jnp.float32),
            pltpu.VMEM((C,), jnp.int32),
            pltpu.VMEM((C,), jnp.int32),
            pltpu.VMEM((C,), jnp.int32),
            pltpu.VMEM((C,), jnp.int32),
            pltpu.VMEM((C,), jnp.int32),
            pltpu.VMEM((C,), jnp.int32),
            pltpu.VMEM((C,), jnp.int32),
            pltpu.VMEM((L,), jnp.float32),
            pltpu.VMEM_SHARED((NUM_SPECIES, SPECIES_D), jnp.float32),
            pltpu.VMEM_SHARED((NS, C * OUT_D), jnp.float32),
            pltpu.SemaphoreType.DMA,
            pltpu.SemaphoreType.DMA,
            pltpu.SemaphoreType.DMA,
            pltpu.SemaphoreType.DMA,
        ],
    )(_body)
    return f(xf, g, sp, ab, it, mv)


def kernel(x, group_idx, species_emb, ability_emb, item_emb, move_emb):
    xf = x.reshape(N * X_D)
    g = jnp.full((L,), group_idx, jnp.float32)
    sp = species_emb.at[0].set(0.0)
    out = _run(xf, g, sp, ability_emb, item_emb, move_emb)
    return out.reshape(x.shape[0], x.shape[1], OUT_D)


# speedup vs baseline: 1.0723x; 1.0723x over previous
import functools

import jax
import jax.numpy as jnp
from jax import lax
from jax.experimental import pallas as pl
from jax.experimental.pallas import tpu as pltpu
from jax.experimental.pallas import tpu_sc as plsc

L = 16
NC, NS = 2, 16
NW = NC * NS
N = 1024 * 200
C = 128
PER_W = N // NW
NCHUNK = PER_W // C

X_D = 32
OUT_D = 153
SPECIES_D = 32
EMB_D = 16

NUM_SPECIES = 2048
NUM_ABILITIES = 512
NUM_ITEMS = 2048
NUM_MOVES = 1024

VOCABS = (NUM_SPECIES, NUM_ABILITIES, NUM_ITEMS,
          NUM_MOVES, NUM_MOVES, NUM_MOVES, NUM_MOVES)
RAW_OFF = 121


def _body(x_hbm, g_hbm, sp_hbm, ab_hbm, it_hbm, mv_hbm, out_hbm,
          ab_t, it_t, mv_t, x_a, x_b, sp_v, out_a, out_b,
          ids0, ids1, ids2, ids3, ids4, ids5, ids6,
          g_v, sp_sh, sem_xa, sem_xb, sem_oa, sem_ob, sem):
    sid = lax.axis_index("s")
    wid = sid * NC + lax.axis_index("c")
    base0 = wid * PER_W

    pltpu.sync_copy(ab_hbm, ab_t)
    pltpu.sync_copy(it_hbm, it_t)
    pltpu.sync_copy(mv_hbm, mv_t)
    pltpu.sync_copy(g_hbm, g_v)
    zrow = jnp.zeros((L,), jnp.float32)
    ab_t[0, :] = zrow
    it_t[0, :] = zrow
    mv_t[0, :] = zrow
    g = g_v[...]

    srows = NUM_SPECIES // NS
    pltpu.sync_copy(sp_hbm.at[pl.ds(sid * srows, srows)], sp_v)
    pltpu.sync_copy(sp_v, sp_sh.at[pl.ds(sid * srows, srows)])
    plsc.subcore_barrier()

    id_bufs = (ids0, ids1, ids2, ids3, ids4, ids5, ids6)
    iota = lax.broadcasted_iota(jnp.int32, (L,), 0)
    col_tabs = ((1, 32, ab_t), (2, 48, it_t), (3, 64, mv_t),
                (4, 80, mv_t), (5, 96, mv_t), (6, 112, mv_t))

    def x_slice(base):
        return x_hbm.at[pl.ds(base * X_D, C * X_D)]

    def out_slice(base):
        return out_hbm.at[pl.ds(base * OUT_D, C * OUT_D)]

    def phase(j, x_p, x_q, out_p, sem_xp, sem_xq, sem_op):
        base = base0 + j * C
        pltpu.make_async_copy(x_slice(base), x_p, sem_xp).wait()

        @pl.when(j + 1 < NCHUNK)
        def _():
            pltpu.async_copy(x_slice(base + C), x_q, sem_xq)

        def ids_blk(b, carry2):
            rvec = iota * X_D + b * (L * X_D)
            for col in range(7):
                xc = plsc.load_gather(x_p, [rvec + col])
                idv = jnp.clip((xc + g).astype(jnp.int32), 0, VOCABS[col] - 1)
                id_bufs[col][pl.ds(b * L, L)] = idv
            return carry2

        lax.fori_loop(0, C // L, ids_blk, 0, unroll=True)

        pltpu.async_copy(sp_sh.at[ids0], sp_v, sem)

        @pl.when(j >= 2)
        def _():
            pltpu.make_async_copy(out_p, out_slice(base - 2 * C), sem_op).wait()

        def asm_blk(b, carry2):
            r_out = (iota + b * L) * OUT_D
            for col, off, tab in col_tabs:
                ids = id_bufs[col][pl.ds(b * L, L)]
                for d in range(EMB_D):
                    v = plsc.load_gather(tab, [ids, jnp.full((L,), d, jnp.int32)])
                    plsc.store_scatter(out_p, [r_out + (off + d)], v)
            return carry2

        lax.fori_loop(0, C // L, asm_blk, 0, unroll=False)
        pltpu.make_async_copy(sp_sh.at[ids0], sp_v, sem).wait()

        def asm_row(r, carry2):
            ro = r * OUT_D
            rx = r * X_D
            out_p[pl.ds(ro, L)] = sp_v[r, 0:L]
            out_p[pl.ds(ro + L, L)] = sp_v[r, L:SPECIES_D]
            out_p[pl.ds(ro + 128, L)] = x_p[pl.ds(rx + 7, L)] + g
            out_p[pl.ds(ro + 137, L)] = x_p[pl.ds(rx + 16, L)] + g
            return carry2

        lax.fori_loop(0, C, asm_row, 0, unroll=4)
        pltpu.async_copy(out_p, out_slice(base), sem_op)

    pltpu.async_copy(x_slice(base0), x_a, sem_xa)

    def pair(k, carry):
        j = 2 * k
        phase(j, x_a, x_b, out_a, sem_xa, sem_xb, sem_oa)
        phase(j + 1, x_b, x_a, out_b, sem_xb, sem_xa, sem_ob)
        return carry

    lax.fori_loop(0, NCHUNK // 2, pair, 0, unroll=False)

    last = base0 + (NCHUNK - 2) * C
    pltpu.make_async_copy(out_a, out_slice(last), sem_oa).wait()
    pltpu.make_async_copy(out_b, out_slice(last + C), sem_ob).wait()


@functools.partial(jax.jit)
def _run(xf, g, sp, ab, it, mv):
    mesh = plsc.VectorSubcoreMesh(core_axis_name="c", subcore_axis_name="s")
    f = functools.partial(
        pl.kernel,
        mesh=mesh,
        compiler_params=pltpu.CompilerParams(
            needs_layout_passes=False, use_tc_tiling_on_sc=False),
        out_type=jax.ShapeDtypeStruct((N * OUT_D,), jnp.float32),
        scratch_types=[
            pltpu.VMEM((NUM_ABILITIES, EMB_D), jnp.float32),
            pltpu.VMEM((NUM_ITEMS, EMB_D), jnp.float32),
            pltpu.VMEM((NUM_MOVES, EMB_D), jnp.float32),
            pltpu.VMEM((C * X_D,), jnp.float32),
            pltpu.VMEM((C * X_D,), jnp.float32),
            pltpu.VMEM((C, SPECIES_D), jnp.float32),
            pltpu.VMEM((C * OUT_D,), jnp.float32),
            pltpu.VMEM((C * OUT_D,), jnp.float32),
            pltpu.VMEM((C,), jnp.int32),
            pltpu.VMEM((C,), jnp.int32),
            pltpu.VMEM((C,), jnp.int32),
            pltpu.VMEM((C,), jnp.int32),
            pltpu.VMEM((C,), jnp.int32),
            pltpu.VMEM((C,), jnp.int32),
            pltpu.VMEM((C,), jnp.int32),
            pltpu.VMEM((L,), jnp.float32),
            pltpu.VMEM_SHARED((NUM_SPECIES, SPECIES_D), jnp.float32),
            pltpu.SemaphoreType.DMA,
            pltpu.SemaphoreType.DMA,
            pltpu.SemaphoreType.DMA,
            pltpu.SemaphoreType.DMA,
            pltpu.SemaphoreType.DMA,
        ],
    )(_body)
    return f(xf, g, sp, ab, it, mv)


def kernel(x, group_idx, species_emb, ability_emb, item_emb, move_emb):
    xf = x.reshape(N * X_D)
    g = jnp.full((L,), group_idx, jnp.float32)
    sp = species_emb.at[0].set(0.0)
    out = _run(xf, g, sp, ability_emb, item_emb, move_emb)
    return out.reshape(x.shape[0], x.shape[1], OUT_D)
